# Initial kernel scaffold; baseline (speedup 1.0000x reference)
import jax
import jax.numpy as jnp
from jax.experimental import pallas as pl


def kernel(x, gate_w, w1_v1, w2):
    raise NotImplementedError("write your pallas kernel here")


# speedup vs baseline: 1.8483x; 1.8483x over previous
import functools

import jax
import jax.numpy as jnp
from jax.experimental import pallas as pl


def _router_kernel(x_ref, gw_ref, sc_ref):
    x = x_ref[...]
    gw = gw_ref[...]
    logits = jax.lax.dot_general(
        x, gw, (((1,), (1,)), ((), ())), preferred_element_type=jnp.float32
    )
    m = jnp.max(logits, axis=-1, keepdims=True)
    ex = jnp.exp(logits - m)
    p = ex / jnp.sum(ex, axis=-1, keepdims=True)
    E = p.shape[-1]
    lane = jax.lax.broadcasted_iota(jnp.int32, p.shape, 1)
    max1 = jnp.max(p, axis=-1, keepdims=True)
    is1 = p == max1
    idx1 = jnp.min(jnp.where(is1, lane, E), axis=-1, keepdims=True)
    sel1 = lane == idx1
    p_wo = jnp.where(sel1, -jnp.inf, p)
    max2 = jnp.max(p_wo, axis=-1, keepdims=True)
    is2 = p_wo == max2
    idx2 = jnp.min(jnp.where(is2, lane, E), axis=-1, keepdims=True)
    sel2 = lane == idx2
    denom = max1 + max2
    scores = jnp.where(sel1 | sel2, p / denom, 0.0)
    sc_ref[...] = scores.T


def _moe_kernel(x_ref, sc_ref, w1_ref, v1_ref, w2_ref, out_ref):
    e = pl.program_id(0)
    f = pl.program_id(1)

    @pl.when((e == 0) & (f == 0))
    def _init():
        out_ref[...] = jnp.zeros_like(out_ref)

    x = x_ref[...]
    w1 = w1_ref[0]
    v1 = v1_ref[0]
    g = jax.lax.dot_general(x, w1, (((1,), (1,)), ((), ())),
                            preferred_element_type=jnp.float32)
    u = jax.lax.dot_general(x, v1, (((1,), (1,)), ((), ())),
                            preferred_element_type=jnp.float32)
    h = (g * jax.lax.logistic(g)) * u
    s = sc_ref[0, 0, :]
    h = h * s[:, None]
    y = jax.lax.dot_general(h, w2_ref[0], (((1,), (0,)), ((), ())),
                            preferred_element_type=jnp.float32)
    out_ref[...] += y


@jax.jit
def kernel(x, gate_w, w1_v1, w2):
    T, D = x.shape
    E = gate_w.shape[0]
    F = w1_v1.shape[1] // 2
    BF = 512
    NF = F // BF

    scores_t = pl.pallas_call(
        _router_kernel,
        out_shape=jax.ShapeDtypeStruct((E, T), jnp.float32),
    )(x, gate_w)
    scores_t3 = scores_t.reshape(E, 1, T)

    out = pl.pallas_call(
        _moe_kernel,
        grid=(E, NF),
        in_specs=[
            pl.BlockSpec((T, D), lambda e, f: (0, 0)),
            pl.BlockSpec((1, 1, T), lambda e, f: (e, 0, 0)),
            pl.BlockSpec((1, BF, D), lambda e, f: (e, f, 0)),
            pl.BlockSpec((1, BF, D), lambda e, f, NF=NF: (e, f + NF, 0)),
            pl.BlockSpec((1, BF, D), lambda e, f: (e, f, 0)),
        ],
        out_specs=pl.BlockSpec((T, D), lambda e, f: (0, 0)),
        out_shape=jax.ShapeDtypeStruct((T, D), jnp.float32),
    )(x, scores_t3, w1_v1, w1_v1, w2)
    return out


# speedup vs baseline: 2.3293x; 1.2602x over previous
import functools

import jax
import jax.numpy as jnp
from jax import lax
from jax.experimental import pallas as pl
from jax.experimental.pallas import tpu as pltpu
from jax.experimental.pallas import tpu_sc as plsc

T, D, E, F, K = 2048, 768, 8, 2048, 2
BT = 256
NB = T * K // BT + E
ROWS = NB * BT
SCAL_N = 32
NW = 32
CHUNK = T // NW


def _router_body(x_ref, gw_ref, pos0_ref, pos1_ref, w0e_ref, w1e_ref, scal_ref):
    x = x_ref[...]
    gw = gw_ref[...]
    logits = lax.dot_general(x, gw, (((1,), (1,)), ((), ())),
                             preferred_element_type=jnp.float32)
    m = jnp.max(logits, axis=-1, keepdims=True)
    ex = jnp.exp(logits - m)
    p = ex / jnp.sum(ex, axis=-1, keepdims=True)
    lane = lax.broadcasted_iota(jnp.int32, p.shape, 1)
    max1 = jnp.max(p, axis=-1, keepdims=True)
    idx1 = jnp.min(jnp.where(p == max1, lane, E), axis=-1, keepdims=True)
    sel1 = lane == idx1
    p_wo = jnp.where(sel1, -jnp.inf, p)
    max2 = jnp.max(p_wo, axis=-1, keepdims=True)
    idx2 = jnp.min(jnp.where(p_wo == max2, lane, E), axis=-1, keepdims=True)
    sel2 = lane == idx2
    denom = max1 + max2
    M = (sel1 | sel2).astype(jnp.float32)
    r_i = lax.broadcasted_iota(jnp.int32, (T, T), 0)
    c_i = lax.broadcasted_iota(jnp.int32, (T, T), 1)
    Ls = (c_i < r_i).astype(jnp.float32)
    prefix = lax.dot_general(Ls, M, (((1,), (0,)), ((), ())),
                             preferred_element_type=jnp.float32)
    counts = jnp.sum(M, axis=0, keepdims=True)
    ci = counts.astype(jnp.int32)
    cnt_pad = ((ci + BT - 1) // BT) * BT
    e_r = lax.broadcasted_iota(jnp.int32, (E, E), 0)
    e_c = lax.broadcasted_iota(jnp.int32, (E, E), 1)
    A = (e_r < e_c).astype(jnp.float32)
    off = lax.dot_general(cnt_pad.astype(jnp.float32), A,
                          (((1,), (0,)), ((), ())),
                          preferred_element_type=jnp.float32)
    dest = off + prefix
    pos0 = jnp.sum(jnp.where(sel1, dest, 0.0), axis=1, keepdims=True)
    pos1 = jnp.sum(jnp.where(sel2, dest, 0.0), axis=1, keepdims=True)
    pos0_ref[...] = pos0.T.astype(jnp.int32)
    pos1_ref[...] = pos1.T.astype(jnp.int32)
    w0 = jnp.sum(jnp.where(sel1, p, 0.0), axis=1, keepdims=True) / denom
    w1 = jnp.sum(jnp.where(sel2, p, 0.0), axis=1, keepdims=True) / denom
    w0e_ref[...] = jnp.broadcast_to(w0, (T, 16))
    w1e_ref[...] = jnp.broadcast_to(w1, (T, 16))
    off_i = off.astype(jnp.int32)
    bb = lax.broadcasted_iota(jnp.int32, (E, SCAL_N), 1)
    cmp = (bb * BT >= off_i.T).astype(jnp.int32)
    be = jnp.sum(cmp, axis=0, keepdims=True) - 1
    nbv = jnp.sum(cnt_pad, axis=1, keepdims=True) // BT
    s_lane = lax.broadcasted_iota(jnp.int32, (1, SCAL_N), 1)
    scal_ref[...] = jnp.where(s_lane == NB, nbv, be)


def _ffn_body(scal_ref, xs_ref, w1_ref, v1_ref, w2_ref, ys_ref):
    b = pl.program_id(0)
    nbv = scal_ref[NB]

    @pl.when(b < nbv)
    def _():
        xs = xs_ref[...]
        g = lax.dot_general(xs, w1_ref[0], (((1,), (1,)), ((), ())),
                            preferred_element_type=jnp.float32)
        u = lax.dot_general(xs, v1_ref[0], (((1,), (1,)), ((), ())),
                            preferred_element_type=jnp.float32)
        h = (g * lax.logistic(g)) * u
        ys_ref[...] = lax.dot_general(h, w2_ref[0], (((1,), (0,)), ((), ())),
                                      preferred_element_type=jnp.float32)


def _dispatch_body(x_hbm, pos0_hbm, pos1_hbm, xs_hbm,
                   xbuf, idx0, idx1, sem0, sem1):
    wid = lax.axis_index("s") * 2 + lax.axis_index("c")
    base = wid * CHUNK
    pltpu.sync_copy(x_hbm.at[pl.ds(base, CHUNK)], xbuf)
    pltpu.sync_copy(pos0_hbm.at[pl.ds(base, CHUNK)], idx0)
    pltpu.sync_copy(pos1_hbm.at[pl.ds(base, CHUNK)], idx1)
    c0 = pltpu.async_copy(xbuf, xs_hbm.at[idx0], sem0)
    c1 = pltpu.async_copy(xbuf, xs_hbm.at[idx1], sem1)
    c0.wait()
    c1.wait()


def _combine_body(ys_hbm, pos0_hbm, pos1_hbm, w0_hbm, w1_hbm, out_hbm,
                  y0buf, y1buf, idx0, idx1, w0buf, w1buf, sem0, sem1):
    wid = lax.axis_index("s") * 2 + lax.axis_index("c")
    base = wid * CHUNK
    pltpu.sync_copy(pos0_hbm.at[pl.ds(base, CHUNK)], idx0)
    pltpu.sync_copy(pos1_hbm.at[pl.ds(base, CHUNK)], idx1)
    pltpu.sync_copy(w0_hbm.at[pl.ds(base * 16, CHUNK * 16)], w0buf)
    pltpu.sync_copy(w1_hbm.at[pl.ds(base * 16, CHUNK * 16)], w1buf)
    c0 = pltpu.async_copy(ys_hbm.at[idx0], y0buf, sem0)
    c1 = pltpu.async_copy(ys_hbm.at[idx1], y1buf, sem1)
    c0.wait()
    c1.wait()

    def body(t, carry):
        w0v = w0buf[pl.ds(t * 16, 16)]
        w1v = w1buf[pl.ds(t * 16, 16)]
        for c in range(D // 16):
            sl = pl.ds(c * 16, 16)
            y0buf[t, sl] = w0v * y0buf[t, sl] + w1v * y1buf[t, sl]
        return carry

    lax.fori_loop(0, CHUNK, body, 0)
    pltpu.sync_copy(y0buf, out_hbm.at[pl.ds(base, CHUNK)])


@functools.cache
def _sc_kernels():
    mesh = plsc.VectorSubcoreMesh(core_axis_name="c", subcore_axis_name="s")
    dispatch = pl.kernel(
        _dispatch_body,
        out_type=jax.ShapeDtypeStruct((ROWS, D), jnp.float32),
        mesh=mesh,
        scratch_types=[
            pltpu.VMEM((CHUNK, D), jnp.float32),
            pltpu.VMEM((CHUNK,), jnp.int32),
            pltpu.VMEM((CHUNK,), jnp.int32),
            pltpu.SemaphoreType.DMA,
            pltpu.SemaphoreType.DMA,
        ],
    )
    combine = pl.kernel(
        _combine_body,
        out_type=jax.ShapeDtypeStruct((T, D), jnp.float32),
        mesh=mesh,
        scratch_types=[
            pltpu.VMEM((CHUNK, D), jnp.float32),
            pltpu.VMEM((CHUNK, D), jnp.float32),
            pltpu.VMEM((CHUNK,), jnp.int32),
            pltpu.VMEM((CHUNK,), jnp.int32),
            pltpu.VMEM((CHUNK * 16,), jnp.float32),
            pltpu.VMEM((CHUNK * 16,), jnp.float32),
            pltpu.SemaphoreType.DMA,
            pltpu.SemaphoreType.DMA,
        ],
    )
    return dispatch, combine


@jax.jit
def kernel(x, gate_w, w1_v1, w2):
    pos0, pos1, w0e, w1e, scal = pl.pallas_call(
        _router_body,
        out_shape=[
            jax.ShapeDtypeStruct((1, T), jnp.int32),
            jax.ShapeDtypeStruct((1, T), jnp.int32),
            jax.ShapeDtypeStruct((T, 16), jnp.float32),
            jax.ShapeDtypeStruct((T, 16), jnp.float32),
            jax.ShapeDtypeStruct((1, SCAL_N), jnp.int32),
        ],
    )(x, gate_w)
    pos0 = pos0.reshape(T)
    pos1 = pos1.reshape(T)
    scal = scal.reshape(SCAL_N)
    w0f = w0e.reshape(T * 16)
    w1f = w1e.reshape(T * 16)

    _dispatch, _combine = _sc_kernels()
    xs = _dispatch(x, pos0, pos1)

    ys = pl.pallas_call(
        _ffn_body,
        grid_spec=pltpu.PrefetchScalarGridSpec(
            num_scalar_prefetch=1,
            grid=(NB,),
            in_specs=[
                pl.BlockSpec((BT, D), lambda b, s: (b, 0)),
                pl.BlockSpec((1, F, D), lambda b, s: (s[b], 0, 0)),
                pl.BlockSpec((1, F, D), lambda b, s: (s[b], 1, 0)),
                pl.BlockSpec((1, F, D), lambda b, s: (s[b], 0, 0)),
            ],
            out_specs=pl.BlockSpec((BT, D), lambda b, s: (b, 0)),
        ),
        out_shape=jax.ShapeDtypeStruct((ROWS, D), jnp.float32),
    )(scal, xs, w1_v1, w1_v1, w2)

    out = _combine(ys, pos0, pos1, w0f, w1f)
    return out


# speedup vs baseline: 2.5052x; 1.0755x over previous
import functools

import jax
import jax.numpy as jnp
from jax import lax
from jax.experimental import pallas as pl
from jax.experimental.pallas import tpu as pltpu
from jax.experimental.pallas import tpu_sc as plsc

T, D, E, F, K = 2048, 768, 8, 2048, 2
BT = 512
NB = T * K // BT + E
ROWS = NB * BT
SCAL_N = 32
NW = 32
CHUNK = T // NW


def _router_body(x_ref, gw_ref, pos0_ref, pos1_ref, w0e_ref, w1e_ref, scal_ref):
    x = x_ref[...]
    gw = gw_ref[...]
    logits = lax.dot_general(x, gw, (((1,), (1,)), ((), ())),
                             preferred_element_type=jnp.float32)
    m = jnp.max(logits, axis=-1, keepdims=True)
    ex = jnp.exp(logits - m)
    p = ex / jnp.sum(ex, axis=-1, keepdims=True)
    lane = lax.broadcasted_iota(jnp.int32, p.shape, 1)
    max1 = jnp.max(p, axis=-1, keepdims=True)
    idx1 = jnp.min(jnp.where(p == max1, lane, E), axis=-1, keepdims=True)
    sel1 = lane == idx1
    p_wo = jnp.where(sel1, -jnp.inf, p)
    max2 = jnp.max(p_wo, axis=-1, keepdims=True)
    idx2 = jnp.min(jnp.where(p_wo == max2, lane, E), axis=-1, keepdims=True)
    sel2 = lane == idx2
    denom = max1 + max2
    M = (sel1 | sel2).astype(jnp.float32)
    r_i = lax.broadcasted_iota(jnp.int32, (T, T), 0)
    c_i = lax.broadcasted_iota(jnp.int32, (T, T), 1)
    Ls = (c_i < r_i).astype(jnp.float32)
    prefix = lax.dot_general(Ls, M, (((1,), (0,)), ((), ())),
                             preferred_element_type=jnp.float32)
    counts = jnp.sum(M, axis=0, keepdims=True)
    ci = counts.astype(jnp.int32)
    cnt_pad = ((ci + BT - 1) // BT) * BT
    e_r = lax.broadcasted_iota(jnp.int32, (E, E), 0)
    e_c = lax.broadcasted_iota(jnp.int32, (E, E), 1)
    A = (e_r < e_c).astype(jnp.float32)
    off = lax.dot_general(cnt_pad.astype(jnp.float32), A,
                          (((1,), (0,)), ((), ())),
                          preferred_element_type=jnp.float32)
    dest = off + prefix
    pos0 = jnp.sum(jnp.where(sel1, dest, 0.0), axis=1, keepdims=True)
    pos1 = jnp.sum(jnp.where(sel2, dest, 0.0), axis=1, keepdims=True)
    pos0_ref[...] = pos0.T.astype(jnp.int32)
    pos1_ref[...] = pos1.T.astype(jnp.int32)
    w0 = jnp.sum(jnp.where(sel1, p, 0.0), axis=1, keepdims=True) / denom
    w1 = jnp.sum(jnp.where(sel2, p, 0.0), axis=1, keepdims=True) / denom
    w0e_ref[...] = jnp.broadcast_to(w0, (T, 16))
    w1e_ref[...] = jnp.broadcast_to(w1, (T, 16))
    off_i = off.astype(jnp.int32)
    bb = lax.broadcasted_iota(jnp.int32, (E, SCAL_N), 1)
    cmp = (bb * BT >= off_i.T).astype(jnp.int32)
    be = jnp.sum(cmp, axis=0, keepdims=True) - 1
    nbv = jnp.sum(cnt_pad, axis=1, keepdims=True) // BT
    s_lane = lax.broadcasted_iota(jnp.int32, (1, SCAL_N), 1)
    scal_ref[...] = jnp.where(s_lane == NB, nbv, be)


def _ffn_body(scal_ref, xs_ref, w1_ref, v1_ref, w2_ref, ys_ref):
    b = pl.program_id(0)
    nbv = scal_ref[NB]

    @pl.when(b < nbv)
    def _():
        xs = xs_ref[...]
        g = lax.dot_general(xs, w1_ref[0], (((1,), (1,)), ((), ())),
                            preferred_element_type=jnp.float32)
        u = lax.dot_general(xs, v1_ref[0], (((1,), (1,)), ((), ())),
                            preferred_element_type=jnp.float32)
        h = (g * lax.logistic(g)) * u
        ys_ref[...] = lax.dot_general(h, w2_ref[0], (((1,), (0,)), ((), ())),
                                      preferred_element_type=jnp.float32)


def _dispatch_body(x_hbm, pos0_hbm, pos1_hbm, xs_hbm,
                   xbuf, idx0, idx1, sem0, sem1):
    wid = lax.axis_index("s") * 2 + lax.axis_index("c")
    base = wid * CHUNK
    pltpu.sync_copy(x_hbm.at[pl.ds(base, CHUNK)], xbuf)
    pltpu.sync_copy(pos0_hbm.at[pl.ds(base, CHUNK)], idx0)
    pltpu.sync_copy(pos1_hbm.at[pl.ds(base, CHUNK)], idx1)
    c0 = pltpu.async_copy(xbuf, xs_hbm.at[idx0], sem0)
    c1 = pltpu.async_copy(xbuf, xs_hbm.at[idx1], sem1)
    c0.wait()
    c1.wait()


def _combine_body(ys_hbm, pos0_hbm, pos1_hbm, w0_hbm, w1_hbm, out_hbm,
                  y0buf, y1buf, idx0, idx1, w0buf, w1buf, sem0, sem1):
    wid = lax.axis_index("s") * 2 + lax.axis_index("c")
    base = wid * CHUNK
    pltpu.sync_copy(pos0_hbm.at[pl.ds(base, CHUNK)], idx0)
    pltpu.sync_copy(pos1_hbm.at[pl.ds(base, CHUNK)], idx1)
    pltpu.sync_copy(w0_hbm.at[pl.ds(base * 16, CHUNK * 16)], w0buf)
    pltpu.sync_copy(w1_hbm.at[pl.ds(base * 16, CHUNK * 16)], w1buf)
    c0 = pltpu.async_copy(ys_hbm.at[idx0], y0buf, sem0)
    c1 = pltpu.async_copy(ys_hbm.at[idx1], y1buf, sem1)
    c0.wait()
    c1.wait()

    def body(t, carry):
        w0v = w0buf[pl.ds(t * 16, 16)]
        w1v = w1buf[pl.ds(t * 16, 16)]
        for c in range(D // 16):
            sl = pl.ds(c * 16, 16)
            y0buf[t, sl] = w0v * y0buf[t, sl] + w1v * y1buf[t, sl]
        return carry

    lax.fori_loop(0, CHUNK, body, 0)
    pltpu.sync_copy(y0buf, out_hbm.at[pl.ds(base, CHUNK)])


@functools.cache
def _sc_kernels():
    mesh = plsc.VectorSubcoreMesh(core_axis_name="c", subcore_axis_name="s")
    dispatch = pl.kernel(
        _dispatch_body,
        out_type=jax.ShapeDtypeStruct((ROWS, D), jnp.float32),
        mesh=mesh,
        scratch_types=[
            pltpu.VMEM((CHUNK, D), jnp.float32),
            pltpu.VMEM((CHUNK,), jnp.int32),
            pltpu.VMEM((CHUNK,), jnp.int32),
            pltpu.SemaphoreType.DMA,
            pltpu.SemaphoreType.DMA,
        ],
    )
    combine = pl.kernel(
        _combine_body,
        out_type=jax.ShapeDtypeStruct((T, D), jnp.float32),
        mesh=mesh,
        scratch_types=[
            pltpu.VMEM((CHUNK, D), jnp.float32),
            pltpu.VMEM((CHUNK, D), jnp.float32),
            pltpu.VMEM((CHUNK,), jnp.int32),
            pltpu.VMEM((CHUNK,), jnp.int32),
            pltpu.VMEM((CHUNK * 16,), jnp.float32),
            pltpu.VMEM((CHUNK * 16,), jnp.float32),
            pltpu.SemaphoreType.DMA,
            pltpu.SemaphoreType.DMA,
        ],
    )
    return dispatch, combine


@jax.jit
def kernel(x, gate_w, w1_v1, w2):
    pos0, pos1, w0e, w1e, scal = pl.pallas_call(
        _router_body,
        out_shape=[
            jax.ShapeDtypeStruct((1, T), jnp.int32),
            jax.ShapeDtypeStruct((1, T), jnp.int32),
            jax.ShapeDtypeStruct((T, 16), jnp.float32),
            jax.ShapeDtypeStruct((T, 16), jnp.float32),
            jax.ShapeDtypeStruct((1, SCAL_N), jnp.int32),
        ],
    )(x, gate_w)
    pos0 = pos0.reshape(T)
    pos1 = pos1.reshape(T)
    scal = scal.reshape(SCAL_N)
    w0f = w0e.reshape(T * 16)
    w1f = w1e.reshape(T * 16)

    _dispatch, _combine = _sc_kernels()
    xs = _dispatch(x, pos0, pos1)

    ys = pl.pallas_call(
        _ffn_body,
        grid_spec=pltpu.PrefetchScalarGridSpec(
            num_scalar_prefetch=1,
            grid=(NB,),
            in_specs=[
                pl.BlockSpec((BT, D), lambda b, s: (b, 0)),
                pl.BlockSpec((1, F, D), lambda b, s: (s[b], 0, 0)),
                pl.BlockSpec((1, F, D), lambda b, s: (s[b], 1, 0)),
                pl.BlockSpec((1, F, D), lambda b, s: (s[b], 0, 0)),
            ],
            out_specs=pl.BlockSpec((BT, D), lambda b, s: (b, 0)),
        ),
        out_shape=jax.ShapeDtypeStruct((ROWS, D), jnp.float32),
    )(scal, xs, w1_v1, w1_v1, w2)

    out = _combine(ys, pos0, pos1, w0f, w1f)
    return out
